# baseline (speedup 1.0000x reference)
import functools

import jax
import jax.numpy as jnp
from jax import lax
from jax.experimental import pallas as pl
from jax.experimental.pallas import tpu as pltpu
from jax.experimental.pallas import tpu_sc as plsc

_SC_RING = 3
_TC_RING = 8


def _sc_body(n_samples, planes, hist_ref, out_ref, buf, in_sem, out_sem):
    nc = 2
    wid = lax.axis_index("s") * nc + lax.axis_index("c")
    H = 5
    C = hist_ref.shape[1]
    row = H * (wid // (H - 1)) + wid % (H - 1) + 1
    n_units = C // planes

    def start_in(i):
        pltpu.make_async_copy(
            hist_ref.at[pl.ds(row, 1), pl.ds(i * planes, planes)],
            buf.at[pl.ds(i % _SC_RING, 1)],
            in_sem.at[i % _SC_RING],
        ).start()

    def wait_in(i):
        pltpu.make_async_copy(
            hist_ref.at[pl.ds(row, 1), pl.ds(i * planes, planes)],
            buf.at[pl.ds(i % _SC_RING, 1)],
            in_sem.at[i % _SC_RING],
        ).wait()

    def start_out(i):
        pltpu.make_async_copy(
            buf.at[pl.ds(i % _SC_RING, 1)],
            out_ref.at[pl.ds(row, 1), pl.ds(i * planes, planes)],
            out_sem.at[i % _SC_RING],
        ).start()

    def wait_out(i):
        pltpu.make_async_copy(
            buf.at[pl.ds(i % _SC_RING, 1)],
            out_ref.at[pl.ds(row, 1), pl.ds(i * planes, planes)],
            out_sem.at[i % _SC_RING],
        ).wait()

    for i in range(_SC_RING):
        start_in(i)
    for i in range(n_units):
        wait_in(i)
        start_out(i)
        j = i + _SC_RING
        if j < n_units:
            wait_out(i)
            start_in(j)
    for i in range(n_units - _SC_RING, n_units):
        wait_out(i)


def _tc_body(starts_ref, curr_ref, partial_ref, out_ref, buf, in_sem, out_sem):
    del partial_ref
    B = curr_ref.shape[0]
    H = 5

    def start_in(i):
        pltpu.make_async_copy(
            curr_ref.at[pl.ds(starts_ref[i], 1)],
            buf.at[pl.ds(i % _TC_RING, 1)],
            in_sem.at[i],
        ).start()

    def wait_in(i):
        pltpu.make_async_copy(
            curr_ref.at[pl.ds(starts_ref[i], 1)],
            buf.at[pl.ds(i % _TC_RING, 1)],
            in_sem.at[i],
        ).wait()

    def start_out(i):
        pltpu.make_async_copy(
            buf.at[pl.ds(i % _TC_RING, 1)], out_ref.at[pl.ds(i * H, 1)], out_sem.at[i]
        ).start()

    def wait_out(i):
        pltpu.make_async_copy(
            buf.at[pl.ds(i % _TC_RING, 1)], out_ref.at[pl.ds(i * H, 1)], out_sem.at[i]
        ).wait()

    for i in range(_TC_RING):
        start_in(i)
    for i in range(B):
        wait_in(i)
        start_out(i)
        j = i + _TC_RING
        if j < B:
            wait_out(i)
            start_in(j)
    for i in range(B - _TC_RING, B):
        wait_out(i)


def kernel(feat_curr, feat_history, record_len):
    B, H, C, Hh, W = feat_history.shape
    n_rows = B * H
    n_workers = 32
    planes = C // n_workers
    starts = (jnp.cumsum(record_len) - record_len).astype(jnp.int32)

    hist_tbl = feat_history.reshape(n_rows, C, Hh, W)

    mesh = plsc.VectorSubcoreMesh(core_axis_name="c", subcore_axis_name="s")
    sc_copy = pl.kernel(
        functools.partial(_sc_body, B, planes),
        mesh=mesh,
        out_type=jax.ShapeDtypeStruct((n_rows, C, Hh, W), feat_curr.dtype),
        scratch_types=[
            pltpu.VMEM((_SC_RING, planes, Hh, W), feat_curr.dtype),
            pltpu.SemaphoreType.DMA((_SC_RING,)),
            pltpu.SemaphoreType.DMA((_SC_RING,)),
        ],
    )
    partial_out = sc_copy(hist_tbl)

    feat_final = pl.pallas_call(
        _tc_body,
        in_specs=[
            pl.BlockSpec(memory_space=pltpu.SMEM),
            pl.BlockSpec(memory_space=pltpu.MemorySpace.HBM),
            pl.BlockSpec(memory_space=pltpu.MemorySpace.HBM),
        ],
        out_specs=pl.BlockSpec(memory_space=pltpu.MemorySpace.HBM),
        out_shape=jax.ShapeDtypeStruct((n_rows, C, Hh, W), feat_curr.dtype),
        input_output_aliases={2: 0},
        scratch_shapes=[
            pltpu.VMEM((_TC_RING, C, Hh, W), feat_curr.dtype),
            pltpu.SemaphoreType.DMA((B,)),
            pltpu.SemaphoreType.DMA((B,)),
        ],
    )(starts, feat_curr, partial_out)

    offset_loss = jnp.zeros((1,), dtype=feat_final.dtype)
    return (feat_final, offset_loss)


# speedup vs baseline: 1.0035x; 1.0035x over previous
import functools

import jax
import jax.numpy as jnp
from jax import lax
from jax.experimental import pallas as pl
from jax.experimental.pallas import tpu as pltpu
from jax.experimental.pallas import tpu_sc as plsc

_SC_RING = 3
_TC_RING = 4


def _sc_body(n_samples, planes, hist_ref, out_ref, buf, in_sem, out_sem):
    nc = 2
    wid = lax.axis_index("s") * nc + lax.axis_index("c")
    H = 5
    C = hist_ref.shape[1]
    row = H * (wid // (H - 1)) + wid % (H - 1) + 1
    n_units = C // planes

    def start_in(i):
        pltpu.make_async_copy(
            hist_ref.at[pl.ds(row, 1), pl.ds(i * planes, planes)],
            buf.at[pl.ds(i % _SC_RING, 1)],
            in_sem.at[i % _SC_RING],
        ).start()

    def wait_in(i):
        pltpu.make_async_copy(
            hist_ref.at[pl.ds(row, 1), pl.ds(i * planes, planes)],
            buf.at[pl.ds(i % _SC_RING, 1)],
            in_sem.at[i % _SC_RING],
        ).wait()

    def start_out(i):
        pltpu.make_async_copy(
            buf.at[pl.ds(i % _SC_RING, 1)],
            out_ref.at[pl.ds(row, 1), pl.ds(i * planes, planes)],
            out_sem.at[i % _SC_RING],
        ).start()

    def wait_out(i):
        pltpu.make_async_copy(
            buf.at[pl.ds(i % _SC_RING, 1)],
            out_ref.at[pl.ds(row, 1), pl.ds(i * planes, planes)],
            out_sem.at[i % _SC_RING],
        ).wait()

    for i in range(_SC_RING):
        start_in(i)
    for i in range(n_units):
        wait_in(i)
        start_out(i)
        j = i + _SC_RING
        if j < n_units:
            wait_out(i)
            start_in(j)
    for i in range(n_units - _SC_RING, n_units):
        wait_out(i)


def _tc_body(starts_ref, curr_ref, partial_ref, out_ref, buf, in_sem, out_sem):
    del partial_ref
    B = curr_ref.shape[0]
    H = 5

    def start_in(i):
        pltpu.make_async_copy(
            curr_ref.at[pl.ds(starts_ref[i], 1)],
            buf.at[pl.ds(i % _TC_RING, 1)],
            in_sem.at[i],
        ).start()

    def wait_in(i):
        pltpu.make_async_copy(
            curr_ref.at[pl.ds(starts_ref[i], 1)],
            buf.at[pl.ds(i % _TC_RING, 1)],
            in_sem.at[i],
        ).wait()

    def start_out(i):
        pltpu.make_async_copy(
            buf.at[pl.ds(i % _TC_RING, 1)], out_ref.at[pl.ds(i * H, 1)], out_sem.at[i]
        ).start()

    def wait_out(i):
        pltpu.make_async_copy(
            buf.at[pl.ds(i % _TC_RING, 1)], out_ref.at[pl.ds(i * H, 1)], out_sem.at[i]
        ).wait()

    for i in range(_TC_RING):
        start_in(i)
    for i in range(B):
        wait_in(i)
        start_out(i)
        j = i + _TC_RING
        if j < B:
            wait_out(i)
            start_in(j)
    for i in range(B - _TC_RING, B):
        wait_out(i)


def kernel(feat_curr, feat_history, record_len):
    B, H, C, Hh, W = feat_history.shape
    n_rows = B * H
    n_workers = 32
    planes = C // n_workers
    starts = (jnp.cumsum(record_len) - record_len).astype(jnp.int32)

    hist_tbl = feat_history.reshape(n_rows, C, Hh, W)

    mesh = plsc.VectorSubcoreMesh(core_axis_name="c", subcore_axis_name="s")
    sc_copy = pl.kernel(
        functools.partial(_sc_body, B, planes),
        mesh=mesh,
        out_type=jax.ShapeDtypeStruct((n_rows, C, Hh, W), feat_curr.dtype),
        scratch_types=[
            pltpu.VMEM((_SC_RING, planes, Hh, W), feat_curr.dtype),
            pltpu.SemaphoreType.DMA((_SC_RING,)),
            pltpu.SemaphoreType.DMA((_SC_RING,)),
        ],
    )
    partial_out = sc_copy(hist_tbl)

    feat_final = pl.pallas_call(
        _tc_body,
        in_specs=[
            pl.BlockSpec(memory_space=pltpu.SMEM),
            pl.BlockSpec(memory_space=pltpu.MemorySpace.HBM),
            pl.BlockSpec(memory_space=pltpu.MemorySpace.HBM),
        ],
        out_specs=pl.BlockSpec(memory_space=pltpu.MemorySpace.HBM),
        out_shape=jax.ShapeDtypeStruct((n_rows, C, Hh, W), feat_curr.dtype),
        input_output_aliases={2: 0},
        scratch_shapes=[
            pltpu.VMEM((_TC_RING, C, Hh, W), feat_curr.dtype),
            pltpu.SemaphoreType.DMA((B,)),
            pltpu.SemaphoreType.DMA((B,)),
        ],
    )(starts, feat_curr, partial_out)

    offset_loss = jnp.zeros((1,), dtype=feat_final.dtype)
    return (feat_final, offset_loss)
